# Initial kernel scaffold; baseline (speedup 1.0000x reference)
import jax
import jax.numpy as jnp
from jax.experimental import pallas as pl


def kernel(input_ids, token_embedding, position_embedding):
    raise NotImplementedError("write your pallas kernel here")


# speedup vs baseline: 3.6861x; 3.6861x over previous
import functools

import jax
import jax.numpy as jnp
from jax import lax
from jax.experimental import pallas as pl
from jax.experimental.pallas import tpu as pltpu
from jax.experimental.pallas import tpu_sc as plsc

B, S, E = 4096, 200, 64
HALF = 100
R = (B * S) // HALF
NC, NS = 2, 16
NW = NC * NS
ROWS_PER_W = R // NW
G = 8
NCHUNK = ROWS_PER_W // G
LANES = 16


def _body(ids_hbm, tok_hbm, pos_hbm, out_hbm, pos_v, idx_v, rows_v, sem_g):
    wid = lax.axis_index("s") * NC + lax.axis_index("c")
    base = wid * ROWS_PER_W
    pltpu.sync_copy(pos_hbm.at[pl.ds(0, S)], pos_v)

    def chunk(c, carry):
        row_off = base + c * G
        pltpu.sync_copy(ids_hbm.at[pl.ds(row_off, G)], idx_v)
        copies = [
            pltpu.async_copy(tok_hbm.at[idx_v.at[j]], rows_v.at[j], sem_g)
            for j in range(G)
        ]
        for cp in copies:
            cp.wait()

        def p_body(p, carry2):
            for cc in range(E // LANES):
                sl = pl.ds(cc * LANES, LANES)
                pv0 = pos_v[p, sl]
                pv1 = pos_v[p + HALF, sl]
                for j in range(G):
                    pv = pv0 if j % 2 == 0 else pv1
                    rows_v[j, p, sl] += pv
            return carry2

        lax.fori_loop(0, HALF, p_body, 0)
        pltpu.sync_copy(rows_v, out_hbm.at[pl.ds(row_off, G)])
        return carry

    lax.fori_loop(0, NCHUNK, chunk, 0)


@functools.partial(
    pl.kernel,
    out_type=jax.ShapeDtypeStruct((R, HALF, E), jnp.float32),
    mesh=plsc.VectorSubcoreMesh(core_axis_name="c", subcore_axis_name="s"),
    scratch_types=[
        pltpu.VMEM((S, E), jnp.float32),
        pltpu.VMEM((G, HALF), jnp.int32),
        pltpu.VMEM((G, HALF, E), jnp.float32),
        pltpu.SemaphoreType.DMA,
    ],
    compiler_params=pltpu.CompilerParams(use_tc_tiling_on_sc=False),
)
def _sc_lookup(ids_hbm, tok_hbm, pos_hbm, out_hbm, pos_v, idx_v, rows_v, sem_g):
    _body(ids_hbm, tok_hbm, pos_hbm, out_hbm, pos_v, idx_v, rows_v, sem_g)


def kernel(input_ids, token_embedding, position_embedding):
    ids2d = input_ids.reshape(R, HALF).astype(jnp.int32)
    out = _sc_lookup(ids2d, token_embedding, position_embedding)
    return out.reshape(B, S, E)


# speedup vs baseline: 4.0685x; 1.1037x over previous
import functools

import jax
import jax.numpy as jnp
from jax import lax
from jax.experimental import pallas as pl
from jax.experimental.pallas import tpu as pltpu
from jax.experimental.pallas import tpu_sc as plsc

B, S, E = 4096, 200, 64
HALF = 100
R = (B * S) // HALF
NC, NS = 2, 16
NW = NC * NS
ROWS_PER_W = R // NW
G = 8
NCHUNK = ROWS_PER_W // G
LANES = 16


def _body(ids_hbm, tok_hbm, pos_hbm, out_hbm,
          pos_v, idx0, idx1, rows0, rows1, sg0, sg1, so0, so1):
    idx = (idx0, idx1)
    rows = (rows0, rows1)
    sg = (sg0, sg1)
    so = (so0, so1)
    wid = lax.axis_index("s") * NC + lax.axis_index("c")
    base = wid * ROWS_PER_W
    pltpu.sync_copy(pos_hbm.at[pl.ds(0, S)], pos_v)

    def load(c, b):
        row_off = base + c * G
        pltpu.sync_copy(ids_hbm.at[pl.ds(row_off, G)], idx[b])
        return [
            pltpu.async_copy(tok_hbm.at[idx[b].at[j]], rows[b].at[j], sg[b])
            for j in range(G)
        ]

    def compute(b):
        def p_body(p, carry):
            for cc in range(E // LANES):
                sl = pl.ds(cc * LANES, LANES)
                pv0 = pos_v[p, sl]
                pv1 = pos_v[p + HALF, sl]
                for j in range(G):
                    pv = pv0 if j % 2 == 0 else pv1
                    rows[b][j, p, sl] += pv
            return carry

        lax.fori_loop(0, HALF, p_body, 0)

    def store(c, b):
        row_off = base + c * G
        return pltpu.async_copy(rows[b], out_hbm.at[pl.ds(row_off, G)], so[b])

    gathers = [None, None]
    outs = [None, None]
    gathers[0] = load(0, 0)
    for c in range(NCHUNK):
        b = c % 2
        nb = (c + 1) % 2
        if c + 1 < NCHUNK:
            if outs[nb] is not None:
                outs[nb].wait()
            gathers[nb] = load(c + 1, nb)
        for cp in gathers[b]:
            cp.wait()
        compute(b)
        outs[b] = store(c, b)
    outs[(NCHUNK - 2) % 2].wait()
    outs[(NCHUNK - 1) % 2].wait()


@functools.partial(
    pl.kernel,
    out_type=jax.ShapeDtypeStruct((R, HALF, E), jnp.float32),
    mesh=plsc.VectorSubcoreMesh(core_axis_name="c", subcore_axis_name="s"),
    scratch_types=[
        pltpu.VMEM((S, E), jnp.float32),
        pltpu.VMEM((G, HALF), jnp.int32),
        pltpu.VMEM((G, HALF), jnp.int32),
        pltpu.VMEM((G, HALF, E), jnp.float32),
        pltpu.VMEM((G, HALF, E), jnp.float32),
        pltpu.SemaphoreType.DMA,
        pltpu.SemaphoreType.DMA,
        pltpu.SemaphoreType.DMA,
        pltpu.SemaphoreType.DMA,
    ],
    compiler_params=pltpu.CompilerParams(use_tc_tiling_on_sc=False),
)
def _sc_lookup(ids_hbm, tok_hbm, pos_hbm, out_hbm,
               pos_v, idx0, idx1, rows0, rows1, sg0, sg1, so0, so1):
    _body(ids_hbm, tok_hbm, pos_hbm, out_hbm,
          pos_v, idx0, idx1, rows0, rows1, sg0, sg1, so0, so1)


def kernel(input_ids, token_embedding, position_embedding):
    ids2d = input_ids.reshape(R, HALF).astype(jnp.int32)
    out = _sc_lookup(ids2d, token_embedding, position_embedding)
    return out.reshape(B, S, E)
